# baseline (speedup 1.0000x reference)
import functools

import jax
import jax.numpy as jnp
from jax import lax
from jax.experimental import pallas as pl
from jax.experimental.pallas import tpu as pltpu
from jax.experimental.pallas import tpu_sc as plsc

_S = 8192
_D = 1024
_SPLIT = 4096

_NC = 2
_NS = 16
_NW = _NC * _NS
_ROWS_PER_W = _SPLIT // _NW
_CHUNK = 32
_NSLOT = 2
_NCHUNK = _ROWS_PER_W // _CHUNK

_mesh = plsc.VectorSubcoreMesh(core_axis_name="c", subcore_axis_name="s")


@functools.partial(
    pl.kernel,
    mesh=_mesh,
    out_type=jax.ShapeDtypeStruct((_SPLIT, _D), jnp.float32),
    scratch_types=(
        [pltpu.VMEM((_NSLOT, _CHUNK, _D), jnp.float32)]
        + [pltpu.SemaphoreType.DMA] * (2 * _NSLOT)
    ),
)
def _sc_copy(table_hbm, out_hbm, buf, *sems):
    sin = sems[:_NSLOT]
    sout = sems[_NSLOT:]
    wid = lax.axis_index("s") * _NC + lax.axis_index("c")
    base = wid * _ROWS_PER_W

    def in_copy(i):
        return pltpu.async_copy(
            table_hbm.at[pl.ds(base + i * _CHUNK, _CHUNK)],
            buf.at[i % _NSLOT], sin[i % _NSLOT])

    def out_copy(i):
        return pltpu.async_copy(
            buf.at[i % _NSLOT],
            out_hbm.at[pl.ds(base + i * _CHUNK, _CHUNK)],
            sout[i % _NSLOT])

    hin = [None] * _NCHUNK
    hout = [None] * _NCHUNK
    hin[0] = in_copy(0)
    for i in range(_NCHUNK):
        if i + 1 < _NCHUNK:
            if i + 1 >= _NSLOT:
                hout[i + 1 - _NSLOT].wait()
            hin[i + 1] = in_copy(i + 1)
        hin[i].wait()
        hout[i] = out_copy(i)
    for i in range(max(0, _NCHUNK - _NSLOT), _NCHUNK):
        hout[i].wait()


_TC_ROWS = _S - _SPLIT
_BLK = 512


def _tc_body(t_ref, o_ref):
    o_ref[...] = t_ref[...]


def kernel(x, table):
    del x
    sc_part = _sc_copy(table)
    tc_part = pl.pallas_call(
        _tc_body,
        grid=(_TC_ROWS // _BLK,),
        in_specs=[pl.BlockSpec((_BLK, _D), lambda i: (i + _SPLIT // _BLK, 0))],
        out_specs=pl.BlockSpec((_BLK, _D), lambda i: (i, 0)),
        out_shape=jax.ShapeDtypeStruct((_TC_ROWS, _D), jnp.float32),
    )(table)
    return jnp.concatenate([sc_part, tc_part], axis=0)[None]


# speedup vs baseline: 1.4563x; 1.4563x over previous
import functools

import jax
import jax.numpy as jnp
from jax import lax
from jax.experimental import pallas as pl
from jax.experimental.pallas import tpu as pltpu
from jax.experimental.pallas import tpu_sc as plsc

_S = 8192
_D = 1024
_NC = 2
_NS = 16
_NW = _NC * _NS
_ROWS_PER_W = _S // _NW
_CHUNK = 32
_NSLOT = 2
_NCHUNK = _ROWS_PER_W // _CHUNK

_mesh = plsc.VectorSubcoreMesh(core_axis_name="c", subcore_axis_name="s")


@functools.partial(
    pl.kernel,
    mesh=_mesh,
    out_type=jax.ShapeDtypeStruct((_S, _D), jnp.float32),
    scratch_types=(
        [pltpu.VMEM_SHARED((_NS, _NSLOT, _CHUNK, _D), jnp.float32)]
        + [pltpu.SemaphoreType.DMA] * (2 * _NSLOT)
    ),
)
def _embed_copy(table_hbm, out_hbm, buf, *sems):
    sin = sems[:_NSLOT]
    sout = sems[_NSLOT:]
    sid = lax.axis_index("s")
    wid = sid * _NC + lax.axis_index("c")
    base = wid * _ROWS_PER_W

    def in_copy(i):
        return pltpu.async_copy(
            table_hbm.at[pl.ds(base + i * _CHUNK, _CHUNK)],
            buf.at[sid, i % _NSLOT], sin[i % _NSLOT])

    def out_copy(i):
        return pltpu.async_copy(
            buf.at[sid, i % _NSLOT],
            out_hbm.at[pl.ds(base + i * _CHUNK, _CHUNK)],
            sout[i % _NSLOT])

    hin = [None] * _NCHUNK
    hout = [None] * _NCHUNK
    hin[0] = in_copy(0)
    for i in range(_NCHUNK):
        if i + 1 < _NCHUNK:
            if i + 1 >= _NSLOT:
                hout[i + 1 - _NSLOT].wait()
            hin[i + 1] = in_copy(i + 1)
        hin[i].wait()
        hout[i] = out_copy(i)
    for i in range(max(0, _NCHUNK - _NSLOT), _NCHUNK):
        hout[i].wait()


def kernel(x, table):
    del x
    return _embed_copy(table)[None]


# speedup vs baseline: 1.4880x; 1.0217x over previous
import functools

import jax
import jax.numpy as jnp
from jax import lax
from jax.experimental import pallas as pl
from jax.experimental.pallas import tpu as pltpu
from jax.experimental.pallas import tpu_sc as plsc

_S = 8192
_D = 1024
_NC = 2
_NS = 16
_NW = _NC * _NS
_ROWS_PER_W = _S // _NW
_CHUNK = 56
_CHUNKS = [56, 56, 56, 56, 32]
_OFFS = [0, 56, 112, 168, 224]
_NCHUNK = len(_CHUNKS)
_NSLOT = 2

_mesh = plsc.VectorSubcoreMesh(core_axis_name="c", subcore_axis_name="s")


@functools.partial(
    pl.kernel,
    mesh=_mesh,
    out_type=jax.ShapeDtypeStruct((_S, _D), jnp.float32),
    scratch_types=(
        [pltpu.VMEM((_NSLOT, _CHUNK, _D), jnp.float32)]
        + [pltpu.SemaphoreType.DMA] * (2 * _NSLOT)
    ),
)
def _embed_copy(table_hbm, out_hbm, buf, *sems):
    sin = sems[:_NSLOT]
    sout = sems[_NSLOT:]
    wid = lax.axis_index("s") * _NC + lax.axis_index("c")
    base = wid * _ROWS_PER_W

    def in_copy(i):
        return pltpu.async_copy(
            table_hbm.at[pl.ds(base + _OFFS[i], _CHUNKS[i])],
            buf.at[i % _NSLOT, pl.ds(0, _CHUNKS[i])], sin[i % _NSLOT])

    def out_copy(i):
        return pltpu.async_copy(
            buf.at[i % _NSLOT, pl.ds(0, _CHUNKS[i])],
            out_hbm.at[pl.ds(base + _OFFS[i], _CHUNKS[i])],
            sout[i % _NSLOT])

    hin = [None] * _NCHUNK
    hout = [None] * _NCHUNK
    hin[0] = in_copy(0)
    for i in range(_NCHUNK):
        if i + 1 < _NCHUNK:
            if i + 1 >= _NSLOT:
                hout[i + 1 - _NSLOT].wait()
            hin[i + 1] = in_copy(i + 1)
        hin[i].wait()
        hout[i] = out_copy(i)
    for i in range(max(0, _NCHUNK - _NSLOT), _NCHUNK):
        hout[i].wait()


def kernel(x, table):
    del x
    return _embed_copy(table)[None]


# speedup vs baseline: 1.4942x; 1.0042x over previous
import functools

import jax
import jax.numpy as jnp
from jax import lax
from jax.experimental import pallas as pl
from jax.experimental.pallas import tpu as pltpu
from jax.experimental.pallas import tpu_sc as plsc

_S = 8192
_D = 1024
_NC = 2
_NS = 16
_NW = _NC * _NS
_ROWS_PER_W = _S // _NW
_CHUNK = 56
_CHUNKS = [56, 56, 56, 56, 32]
_OFFS = [0, 56, 112, 168, 224]
_NCHUNK = len(_CHUNKS)
_NSLOT = 2

_mesh = plsc.VectorSubcoreMesh(core_axis_name="c", subcore_axis_name="s")


@functools.partial(
    pl.kernel,
    mesh=_mesh,
    out_type=jax.ShapeDtypeStruct((_S, _D), jnp.float32),
    scratch_types=(
        [pltpu.VMEM((_NSLOT, _CHUNK, _D), jnp.float32)]
        + [pltpu.SemaphoreType.DMA] * (2 * _NSLOT)
    ),
)
def _embed_copy(table_hbm, out_hbm, buf, *sems):
    sin = sems[:_NSLOT]
    sout = sems[_NSLOT:]
    wid = lax.axis_index("c") * _NS + lax.axis_index("s")
    base = wid * _ROWS_PER_W

    def in_copy(i):
        return pltpu.async_copy(
            table_hbm.at[pl.ds(base + _OFFS[i], _CHUNKS[i])],
            buf.at[i % _NSLOT, pl.ds(0, _CHUNKS[i])], sin[i % _NSLOT])

    def out_copy(i):
        return pltpu.async_copy(
            buf.at[i % _NSLOT, pl.ds(0, _CHUNKS[i])],
            out_hbm.at[pl.ds(base + _OFFS[i], _CHUNKS[i])],
            sout[i % _NSLOT])

    hin = [None] * _NCHUNK
    hout = [None] * _NCHUNK
    hin[0] = in_copy(0)
    for i in range(_NCHUNK):
        if i + 1 < _NCHUNK:
            if i + 1 >= _NSLOT:
                hout[i + 1 - _NSLOT].wait()
            hin[i + 1] = in_copy(i + 1)
        hin[i].wait()
        hout[i] = out_copy(i)
    for i in range(max(0, _NCHUNK - _NSLOT), _NCHUNK):
        hout[i].wait()


def kernel(x, table):
    del x
    return _embed_copy(table)[None]
